# Initial kernel scaffold; baseline (speedup 1.0000x reference)
import jax
import jax.numpy as jnp
from jax.experimental import pallas as pl


def kernel(x, edge_index, W1, b1, W2, b2):
    raise NotImplementedError("write your pallas kernel here")


# speedup vs baseline: 12.9764x; 12.9764x over previous
import functools

import jax
import jax.numpy as jnp
from jax import lax
from jax.experimental import pallas as pl
from jax.experimental.pallas import tpu as pltpu
from jax.experimental.pallas import tpu_sc as plsc

N = 10000
D = 128
E = 320000
NC = 2
NS = 16
NW = NC * NS
CH = 128
NCHUNK = 79
EPT = CH * NCHUNK
EPAD = EPT * NW
NPAD = 10112
RPT = NPAD // NS
DEGR = 80
DRPT = 8

f32 = jnp.float32
i32 = jnp.int32

@functools.lru_cache(maxsize=None)
def _mesh():
    return plsc.VectorSubcoreMesh(
        core_axis_name="c", subcore_axis_name="s", num_cores=NC, num_subcores=NS
    )


def _zero_vmem_rows(ref, nrows):
    zero16 = jnp.zeros((16,), f32)

    def zrow(i, c):
        for k in range(8):
            ref[i, pl.ds(k * 16, 16)] = zero16
        return c

    lax.fori_loop(0, nrows, zrow, 0)


def _deg_body(dst_hbm, deg_out, didx, deg1d, deg2d, rowidx, shared_deg):
    cid = lax.axis_index("c")
    sid = lax.axis_index("s")
    wid = cid * NS + sid

    zero16 = jnp.zeros((16,), f32)

    def z1(i, c):
        deg1d[pl.ds(i * 16, 16)] = zero16
        return c

    lax.fori_loop(0, DEGR * D // 16, z1, 0)
    _zero_vmem_rows(deg2d, DEGR)

    @pl.when(sid == 0)
    def _():
        pltpu.sync_copy(deg2d, shared_deg)

    for k in range(DEGR // 16):
        rowidx[0, pl.ds(k * 16, 16)] = lax.iota(i32, 16) + (16 * k)

    pltpu.sync_copy(dst_hbm.at[wid], didx)
    plsc.subcore_barrier()

    ones16 = jnp.ones((16,), f32)

    def jbody(j, c):
        for k in range(8):
            v = didx[j, pl.ds(k * 16, 16)]
            plsc.addupdate_scatter(deg1d, [v], ones16)
        return c

    lax.fori_loop(0, NCHUNK, jbody, 0)

    def stage(r, c):
        for k in range(8):
            deg2d[r, pl.ds(k * 16, 16)] = deg1d[pl.ds(r * D + k * 16, 16)]
        return c

    lax.fori_loop(0, DEGR, stage, 0)

    pltpu.sync_copy(deg2d, shared_deg.at[rowidx.at[0]], add=True)
    plsc.subcore_barrier()

    @pl.when(sid < DEGR // DRPT)
    def _():
        r0 = sid * DRPT
        pltpu.sync_copy(
            shared_deg.at[pl.ds(r0, DRPT)], deg_out.at[cid].at[pl.ds(r0, DRPT)]
        )


@functools.lru_cache(maxsize=None)
def _deg_kernel():
    return pl.kernel(
        _deg_body,
        out_type=jax.ShapeDtypeStruct((NC, DEGR, D), f32),
        mesh=_mesh(),
        compiler_params=pltpu.CompilerParams(needs_layout_passes=False),
        scratch_types=[
            pltpu.VMEM((NCHUNK, CH), i32),
            pltpu.VMEM((DEGR * D,), f32),
            pltpu.VMEM((DEGR, D), f32),
            pltpu.VMEM((1, DEGR), i32),
            pltpu.VMEM_SHARED((DEGR, D), f32),
        ],
    )


def _agg_body(g_hbm, src_hbm, dst_hbm, acc_out, sidx, didx, rbuf, gsem,
              shared_acc):
    cid = lax.axis_index("c")
    sid = lax.axis_index("s")
    wid = cid * NS + sid

    _zero_vmem_rows(rbuf, CH)

    base = sid * RPT
    nfull = RPT // CH
    for b in range(nfull):
        pltpu.sync_copy(rbuf, shared_acc.at[pl.ds(base + b * CH, CH)])
    rem = RPT - nfull * CH
    pltpu.sync_copy(
        rbuf.at[pl.ds(0, rem)], shared_acc.at[pl.ds(base + nfull * CH, rem)]
    )

    pltpu.sync_copy(src_hbm.at[wid], sidx)
    pltpu.sync_copy(dst_hbm.at[wid], didx)
    plsc.subcore_barrier()

    def jbody(j, c):
        pltpu.async_copy(g_hbm.at[sidx.at[j]], rbuf, gsem).wait()
        pltpu.sync_copy(rbuf, shared_acc.at[didx.at[j]], add=True)
        return c

    lax.fori_loop(0, NCHUNK, jbody, 0)

    plsc.subcore_barrier()
    pltpu.sync_copy(
        shared_acc.at[pl.ds(base, RPT)], acc_out.at[cid].at[pl.ds(base, RPT)]
    )


@functools.lru_cache(maxsize=None)
def _agg_kernel():
    return pl.kernel(
        _agg_body,
        out_type=jax.ShapeDtypeStruct((NC, NPAD, D), f32),
        mesh=_mesh(),
        compiler_params=pltpu.CompilerParams(needs_layout_passes=False),
        scratch_types=[
            pltpu.VMEM((NCHUNK, CH), i32),
            pltpu.VMEM((NCHUNK, CH), i32),
            pltpu.VMEM((CH, D), f32),
            pltpu.SemaphoreType.DMA,
            pltpu.VMEM_SHARED((NPAD, D), f32),
        ],
    )


_BLK = 400
_GRID = N // _BLK


def _dis_body(deg_ref, dis_ref):
    dis_ref[...] = lax.rsqrt(1.0 + deg_ref[0] + deg_ref[1])


def _dis_call(deg2):
    return pl.pallas_call(
        _dis_body,
        out_shape=jax.ShapeDtypeStruct((DEGR, D), f32),
    )(deg2)


def _lin1_body(x_ref, w_ref, b_ref, dis_ref, g_ref, hd_ref):
    h = jnp.dot(x_ref[...], w_ref[...], preferred_element_type=f32)
    h = h + b_ref[...]
    dis = dis_ref[...]
    g_ref[...] = h * dis
    hd_ref[...] = h * (dis * dis)


def _lin1_call(x, W1, b1, dis_col):
    return pl.pallas_call(
        _lin1_body,
        grid=(_GRID,),
        in_specs=[
            pl.BlockSpec((_BLK, D), lambda i: (i, 0)),
            pl.BlockSpec((D, D), lambda i: (0, 0)),
            pl.BlockSpec((1, D), lambda i: (0, 0)),
            pl.BlockSpec((_BLK, 1), lambda i: (i, 0)),
        ],
        out_specs=[
            pl.BlockSpec((_BLK, D), lambda i: (i, 0)),
            pl.BlockSpec((_BLK, D), lambda i: (i, 0)),
        ],
        out_shape=[
            jax.ShapeDtypeStruct((N, D), f32),
            jax.ShapeDtypeStruct((N, D), f32),
        ],
    )(x, W1, b1, dis_col)


def _elu(s):
    return jnp.where(s > 0, s, jnp.exp(s) - 1.0)


def _lin2_body(a_ref, hd_ref, dis_ref, w_ref, b_ref, g_ref, hd2_ref):
    dis = dis_ref[...]
    s = (a_ref[0] + a_ref[1]) * dis + hd_ref[...]
    o = _elu(s)
    h2 = jnp.dot(o, w_ref[...], preferred_element_type=f32) + b_ref[...]
    g_ref[...] = h2 * dis
    hd2_ref[...] = h2 * (dis * dis)


def _lin2_call(acc1, hd1, dis_col, W2, b2):
    return pl.pallas_call(
        _lin2_body,
        grid=(_GRID,),
        in_specs=[
            pl.BlockSpec((NC, _BLK, D), lambda i: (0, i, 0)),
            pl.BlockSpec((_BLK, D), lambda i: (i, 0)),
            pl.BlockSpec((_BLK, 1), lambda i: (i, 0)),
            pl.BlockSpec((D, D), lambda i: (0, 0)),
            pl.BlockSpec((1, D), lambda i: (0, 0)),
        ],
        out_specs=[
            pl.BlockSpec((_BLK, D), lambda i: (i, 0)),
            pl.BlockSpec((_BLK, D), lambda i: (i, 0)),
        ],
        out_shape=[
            jax.ShapeDtypeStruct((N, D), f32),
            jax.ShapeDtypeStruct((N, D), f32),
        ],
    )(acc1, hd1, dis_col, W2, b2)


def _out_body(a_ref, hd_ref, dis_ref, o_ref):
    dis = dis_ref[...]
    o_ref[...] = _elu((a_ref[0] + a_ref[1]) * dis + hd_ref[...])


def _out_call(acc2, hd2, dis_col):
    return pl.pallas_call(
        _out_body,
        grid=(_GRID,),
        in_specs=[
            pl.BlockSpec((NC, _BLK, D), lambda i: (0, i, 0)),
            pl.BlockSpec((_BLK, D), lambda i: (i, 0)),
            pl.BlockSpec((_BLK, 1), lambda i: (i, 0)),
        ],
        out_specs=pl.BlockSpec((_BLK, D), lambda i: (i, 0)),
        out_shape=jax.ShapeDtypeStruct((N, D), f32),
    )(acc2, hd2, dis_col)


def kernel(x, edge_index, W1, b1, W2, b2):
    src = edge_index[0]
    dst = edge_index[1]
    pad = EPAD - E
    srcp = jnp.concatenate([src, jnp.zeros((pad,), i32)]).reshape(NW, NCHUNK, CH)
    dstp = jnp.concatenate([dst, jnp.full((pad,), N, i32)]).reshape(NW, NCHUNK, CH)

    deg2 = _deg_kernel()(dstp)
    dis80 = _dis_call(deg2)
    dis_col = dis80.reshape(-1)[:N].reshape(N, 1)

    g1, hd1 = _lin1_call(x, W1, b1.reshape(1, D), dis_col)
    acc1 = _agg_kernel()(g1, srcp, dstp)
    g2, hd2 = _lin2_call(acc1, hd1, dis_col, W2, b2.reshape(1, D))
    acc2 = _agg_kernel()(g2, srcp, dstp)
    return _out_call(acc2, hd2, dis_col)
